# Initial kernel scaffold; baseline (speedup 1.0000x reference)
import jax
import jax.numpy as jnp
from jax.experimental import pallas as pl


def kernel(x_num, x_cat, emb_tables, W1, b1, g1, be1, W2, b2, g2, be2, W3, b3):
    raise NotImplementedError("write your pallas kernel here")


# speedup vs baseline: 7.8551x; 7.8551x over previous
import functools

import jax
import jax.numpy as jnp
from jax import lax
from jax.experimental import pallas as pl
from jax.experimental.pallas import tpu as pltpu
from jax.experimental.pallas import tpu_sc as plsc

B = 16384
NUM = 13
F = 26
V = 100000
D = 16
H = 128
EPS = 1e-5

NC, NS = 2, 16
NW = NC * NS
BF = B * F
PER_W = BF // NW
RPS = 128
NSTREAM = PER_W // RPS
GRP = 8
NGRP = NSTREAM // GRP


def _sc_gather(table, idx3):
    mesh = plsc.VectorSubcoreMesh(core_axis_name="c", subcore_axis_name="s")

    @functools.partial(
        pl.kernel,
        out_type=jax.ShapeDtypeStruct((BF, D), jnp.float32),
        mesh=mesh,
        compiler_params=pltpu.CompilerParams(use_tc_tiling_on_sc=False),
        scratch_types=[
            pltpu.VMEM((NSTREAM, RPS), jnp.int32),
            pltpu.VMEM((GRP * RPS, D), jnp.float32),
            pltpu.SemaphoreType.DMA,
        ],
    )
    def k(table_hbm, idx_hbm, out_hbm, idx_v, rows_v, sem):
        wid = lax.axis_index("s") * NC + lax.axis_index("c")
        pltpu.sync_copy(idx_hbm.at[wid], idx_v)
        base = wid * PER_W

        def step(g, carry):
            cps = []
            for j in range(GRP):
                cps.append(pltpu.async_copy(
                    table_hbm.at[idx_v.at[g * GRP + j]],
                    rows_v.at[pl.ds(j * RPS, RPS)],
                    sem,
                ))
            for cp in cps:
                cp.wait()
            pltpu.sync_copy(
                rows_v, out_hbm.at[pl.ds(base + g * (GRP * RPS), GRP * RPS)])
            return carry

        lax.fori_loop(0, NGRP, step, 0)

    return k(table, idx3)


def _mlp(emb, x_num, w1e, w1n, b1, w2, b2, w3, b3):
    BLK = 2048

    def body(emb_ref, xn_ref, w1e_ref, w1n_ref, b1_ref, w2_ref, b2_ref,
             w3_ref, b3_ref, o_ref):
        h = jnp.dot(emb_ref[...], w1e_ref[...],
                    preferred_element_type=jnp.float32)
        h = h + jnp.dot(xn_ref[...], w1n_ref[...],
                        preferred_element_type=jnp.float32)
        h = jnp.maximum(h + b1_ref[...], 0.0)
        h = jnp.maximum(
            jnp.dot(h, w2_ref[...], preferred_element_type=jnp.float32)
            + b2_ref[...], 0.0)
        o_ref[...] = (jnp.dot(h, w3_ref[...],
                              preferred_element_type=jnp.float32)
                      + b3_ref[...])

    full = lambda s: pl.BlockSpec(s, lambda i: (0, 0))
    return pl.pallas_call(
        body,
        grid=(B // BLK,),
        in_specs=[
            pl.BlockSpec((BLK, F * D), lambda i: (i, 0)),
            pl.BlockSpec((BLK, NUM), lambda i: (i, 0)),
            full((F * D, H)),
            full((NUM, H)),
            full((1, H)),
            full((H, H // 2)),
            full((1, H // 2)),
            full((H // 2, 1)),
            full((1, 1)),
        ],
        out_specs=pl.BlockSpec((BLK, 1), lambda i: (i, 0)),
        out_shape=jax.ShapeDtypeStruct((B, 1), jnp.float32),
    )(emb, x_num, w1e, w1n, b1, w2, b2, w3, b3)


def kernel(x_num, x_cat, emb_tables, W1, b1, g1, be1, W2, b2, g2, be2, W3, b3):
    table = emb_tables.reshape(F * V, D)
    idx = x_cat.astype(jnp.int32) + (jnp.arange(F, dtype=jnp.int32) * V)[None, :]
    idx3 = idx.reshape(NW, NSTREAM, RPS)
    emb = _sc_gather(table, idx3).reshape(B, F * D)

    inv = 1.0 / jnp.sqrt(1.0 + EPS)
    s1, s2 = g1 * inv, g2 * inv
    w2p = s1[:, None] * W2
    b2p = be1 @ W2 + b2
    w3p = s2[:, None] * W3
    b3p = be2 @ W3 + b3
    return _mlp(emb, x_num, W1[NUM:], W1[:NUM], b1[None, :],
                w2p, b2p[None, :], w3p, b3p[None, :])


# speedup vs baseline: 41.5028x; 5.2836x over previous
import functools

import jax
import jax.numpy as jnp
from jax import lax
from jax.experimental import pallas as pl
from jax.experimental.pallas import tpu as pltpu
from jax.experimental.pallas import tpu_sc as plsc

B = 16384
NUM = 13
F = 26
V = 100000
D = 16
H = 128
EPS = 1e-5

NC, NS = 2, 16
NW = NC * NS
NPAIR = F * D
PAIRS_PER_W = NPAIR // NW
CHUNK = 8192
NCHUNK = B // CHUNK
LANES = 16


def _sc_gather_t(table_t, xcat_t):
    mesh = plsc.VectorSubcoreMesh(core_axis_name="c", subcore_axis_name="s")

    @functools.partial(
        pl.kernel,
        out_type=jax.ShapeDtypeStruct((NPAIR, B), jnp.float32),
        mesh=mesh,
        compiler_params=pltpu.CompilerParams(use_tc_tiling_on_sc=True,
                                             needs_layout_passes=False),
        scratch_types=[
            pltpu.VMEM((V,), jnp.float32),
            pltpu.VMEM((CHUNK,), jnp.int32),
            pltpu.VMEM((CHUNK,), jnp.float32),
        ],
    )
    def k(table_hbm, xcat_hbm, out_hbm, drow_v, idx_v, outc_v):
        wid = lax.axis_index("s") * NC + lax.axis_index("c")
        p0 = wid * PAIRS_PER_W

        def pair_body(j, carry):
            p = p0 + j
            f = p // D
            d = p % D
            pltpu.sync_copy(table_hbm.at[f, d], drow_v)

            def chunk_body(c, cc):
                pltpu.sync_copy(xcat_hbm.at[f, pl.ds(c * CHUNK, CHUNK)], idx_v)

                def t_body(t, tc):
                    sl = pl.ds(t * LANES, LANES)
                    outc_v[sl] = plsc.load_gather(drow_v, [idx_v[sl]])
                    return tc

                lax.fori_loop(0, CHUNK // LANES, t_body, 0)
                pltpu.sync_copy(outc_v,
                                out_hbm.at[p, pl.ds(c * CHUNK, CHUNK)])
                return cc

            lax.fori_loop(0, NCHUNK, chunk_body, 0)
            return carry

        lax.fori_loop(0, PAIRS_PER_W, pair_body, 0)

    return k(table_t, xcat_t)


def _mlp_t(xe_t, xn_t, w1e_t, w1n_t, b1c, w2_t, b2c, w3_t, b3c):
    BLK = 2048

    def body(xe_ref, xn_ref, w1e_ref, w1n_ref, b1_ref, w2_ref, b2_ref,
             w3_ref, b3_ref, o_ref):
        h = jnp.dot(w1e_ref[...], xe_ref[...],
                    preferred_element_type=jnp.float32)
        h = h + jnp.dot(w1n_ref[...], xn_ref[...],
                        preferred_element_type=jnp.float32)
        h = jnp.maximum(h + b1_ref[...], 0.0)
        h = jnp.maximum(
            jnp.dot(w2_ref[...], h, preferred_element_type=jnp.float32)
            + b2_ref[...], 0.0)
        o_ref[...] = (jnp.dot(w3_ref[...], h,
                              preferred_element_type=jnp.float32)
                      + b3_ref[...])

    full = lambda s: pl.BlockSpec(s, lambda i: (0, 0))
    return pl.pallas_call(
        body,
        grid=(B // BLK,),
        in_specs=[
            pl.BlockSpec((NPAIR, BLK), lambda i: (0, i)),
            pl.BlockSpec((NUM, BLK), lambda i: (0, i)),
            full((H, NPAIR)),
            full((H, NUM)),
            full((H, 1)),
            full((H // 2, H)),
            full((H // 2, 1)),
            full((1, H // 2)),
            full((1, 1)),
        ],
        out_specs=pl.BlockSpec((1, BLK), lambda i: (0, i)),
        out_shape=jax.ShapeDtypeStruct((1, B), jnp.float32),
    )(xe_t, xn_t, w1e_t, w1n_t, b1c, w2_t, b2c, w3_t, b3c)


def kernel(x_num, x_cat, emb_tables, W1, b1, g1, be1, W2, b2, g2, be2, W3, b3):
    table_t = emb_tables.transpose(0, 2, 1)
    xcat_t = x_cat.astype(jnp.int32).T
    xn_t = x_num.T

    emb_t = _sc_gather_t(table_t, xcat_t)

    inv = 1.0 / jnp.sqrt(1.0 + EPS)
    s1, s2 = g1 * inv, g2 * inv
    w2p = s1[:, None] * W2
    b2p = be1 @ W2 + b2
    w3p = s2[:, None] * W3
    b3p = be2 @ W3 + b3

    out_t = _mlp_t(emb_t, xn_t, W1[NUM:].T, W1[:NUM].T, b1[:, None],
                   w2p.T, b2p[:, None], w3p.T, b3p[:, None])
    return out_t.reshape(B, 1)
